# Initial kernel scaffold; baseline (speedup 1.0000x reference)
import jax
import jax.numpy as jnp
from jax.experimental import pallas as pl


def kernel(aa_embedding, pos_embedding):
    raise NotImplementedError("write your pallas kernel here")


# speedup vs baseline: 1558.7025x; 1558.7025x over previous
import functools

import jax
import jax.numpy as jnp
from jax import lax
from jax.experimental import pallas as pl
from jax.experimental.pallas import tpu as pltpu
from jax.experimental.pallas import tpu_sc as plsc

MAX_D = 32


def _build_sc_kernel(n):
    info = plsc.get_sparse_core_info()
    nc, ns, lanes = info.num_cores, info.num_subcores, info.num_lanes
    nw = nc * ns
    assert n % nw == 0
    rows_per_w = n // nw
    wlen = n + rows_per_w
    assert wlen % lanes == 0 and rows_per_w % 8 == 0
    n_fill = wlen // lanes
    band_chunks = (2 * MAX_D + 1 + lanes - 1) // lanes + 1
    n_groups = rows_per_w // 8

    mesh = plsc.VectorSubcoreMesh(core_axis_name="c", subcore_axis_name="s")

    @functools.partial(
        pl.kernel,
        mesh=mesh,
        out_type=jax.ShapeDtypeStruct((n, n), jnp.float32),
        scratch_types=[
            pltpu.VMEM((80,), jnp.float32),
            pltpu.VMEM((8, wlen), jnp.float32),
            pltpu.SemaphoreType.DMA,
        ],
        compiler_params=pltpu.CompilerParams(
            use_tc_tiling_on_sc=False, needs_layout_passes=False
        ),
    )
    def k(tbl_hbm, out_hbm, tbl_v, wrev, sem):
        wid = lax.axis_index("s") * nc + lax.axis_index("c")
        r0 = wid * rows_per_w
        base_t = n - rows_per_w - r0

        pltpu.sync_copy(tbl_hbm, tbl_v)

        lane = lax.iota(jnp.int32, lanes)
        cvec = plsc.load_gather(tbl_v, [jnp.full((lanes,), MAX_D, jnp.int32)])

        def fill_body(kk, carry):
            off = pl.multiple_of(kk * lanes, lanes)
            for j in range(8):
                wrev[j, pl.ds(off, lanes)] = cvec
            return carry

        lax.fori_loop(0, n_fill, fill_body, None, unroll=4)

        t0 = n - 1 - MAX_D
        for c in range(band_chunks):
            tvec = t0 + c * lanes + lane
            m = jnp.minimum(jnp.abs(tvec - (n - 1)), MAX_D)
            vals = plsc.load_gather(tbl_v, [m])
            for j in range(8):
                kvec = (t0 - base_t - (7 - j)) + c * lanes + lane
                plsc.store_scatter(
                    wrev, [jnp.full((lanes,), j, jnp.int32), kvec], vals
                )

        copies = []
        for a in range(n_groups):
            k0 = rows_per_w - 8 - 8 * a
            g0 = r0 + 8 * a
            copies.append(
                pltpu.async_copy(
                    wrev.at[:, pl.ds(k0, n)],
                    out_hbm.at[pl.ds(pl.multiple_of(g0, 8), 8)],
                    sem,
                )
            )
        for cp in copies:
            cp.wait()

    return k


@jax.jit
def kernel(aa_embedding, pos_embedding):
    n = aa_embedding.shape[-1]
    tbl = jnp.pad(pos_embedding[:, 0], (0, 80 - pos_embedding.shape[0]))
    out = _build_sc_kernel(n)(tbl)
    return out[None, None]
